# baseline (speedup 1.0000x reference)
import functools

import jax
import jax.numpy as jnp
from jax import lax
from jax.experimental import pallas as pl
from jax.experimental.pallas import tpu as pltpu
from jax.experimental.pallas import tpu_sc as plsc

B = 16384
P = 20
D = 128
NPRED = 64
NARG = 128
NROWS = 648
RB1 = 640
RW2 = 641
RB2 = 642
NIDX = 64
NW = 32
BPW = B // NW
L = 16
DW = D // 2
NCH = D // (2 * L)

PP_OFF = 0
PA_OFF = BPW * P
CP_OFF = PA_OFF + BPW * 2 * P
CA_OFF = CP_OFF + BPW
K_OFF = CA_OFF + 2 * BPW
MBO_OFF = K_OFF + L
IDXN = MBO_OFF + 4 * 3 * L


def _fold_body(pred_ref, arg_ref, w1_ref, b1_ref, w2_ref, b2_ref, out_ref):
    pred = pred_ref[...]
    arg = arg_ref[...]
    w1 = w1_ref[...]
    s = jnp.float32(1.0 / P)
    b2row = jnp.where(
        lax.broadcasted_iota(jnp.int32, (1, D), 1) == 0, b2_ref[...][0], 0.0)
    parts = [
        jnp.dot(pred, w1[0 * D:1 * D], preferred_element_type=jnp.float32) * s,
        jnp.dot(arg, w1[1 * D:2 * D], preferred_element_type=jnp.float32) * s,
        jnp.dot(arg, w1[2 * D:3 * D], preferred_element_type=jnp.float32) * s,
        jnp.dot(pred, w1[3 * D:4 * D], preferred_element_type=jnp.float32),
        jnp.dot(arg, w1[4 * D:5 * D], preferred_element_type=jnp.float32),
        jnp.dot(arg, w1[5 * D:6 * D], preferred_element_type=jnp.float32),
        b1_ref[...][None, :],
        w2_ref[...][:, 0][None, :],
        b2row,
        jnp.zeros((NROWS - RB2 - 1, D), jnp.float32),
    ]
    out_ref[...] = jnp.concatenate(parts, axis=0)


def _pack_pairs(x_f32):
    xb = x_f32.astype(jnp.bfloat16)
    n = xb.shape[-1] // 2
    lo = lax.bitcast_convert_type(xb[:, :n], jnp.uint16).astype(jnp.uint32)
    hi = lax.bitcast_convert_type(xb[:, n:], jnp.uint16).astype(jnp.uint32)
    return lax.bitcast_convert_type(lo | (hi << 16), jnp.int32)


@functools.partial(
    pl.kernel,
    mesh=plsc.VectorSubcoreMesh(core_axis_name="c", subcore_axis_name="s"),
    out_type=jax.ShapeDtypeStruct((B,), jnp.float32),
    compiler_params=pltpu.CompilerParams(needs_layout_passes=False),
    scratch_types=[
        pltpu.VMEM((NROWS * DW,), jnp.int32),
        pltpu.VMEM((IDXN,), jnp.int32),
        pltpu.VMEM((BPW * DW,), jnp.int32),
        pltpu.VMEM((BPW,), jnp.float32),
    ],
)
def _sc_gather(table_hbm, pp_hbm, pa_hbm, cp_hbm, ca_hbm, out_hbm,
               table_v, idx_v, hacc_v, out_v):
    wid = lax.axis_index("s") * 2 + lax.axis_index("c")
    base = wid * BPW
    pltpu.sync_copy(table_hbm, table_v)
    pltpu.sync_copy(pp_hbm.at[pl.ds(base * P, BPW * P)],
                    idx_v.at[pl.ds(PP_OFF, BPW * P)])
    pltpu.sync_copy(pa_hbm.at[pl.ds(base * 2 * P, BPW * 2 * P)],
                    idx_v.at[pl.ds(PA_OFF, BPW * 2 * P)])
    pltpu.sync_copy(cp_hbm.at[pl.ds(base, BPW)], idx_v.at[pl.ds(CP_OFF, BPW)])
    pltpu.sync_copy(ca_hbm.at[pl.ds(base * 2, BPW * 2)],
                    idx_v.at[pl.ds(CA_OFF, BPW * 2)])

    lane = lax.iota(jnp.int32, L)
    idx_v[pl.ds(K_OFF, L)] = jnp.full((L,), RB1, jnp.int32)

    alt = jnp.where(lane % 2 == 0, NPRED, NPRED + NARG)
    zl = jnp.zeros((L,), jnp.int32)
    m3 = jnp.where(lane < 8, 2 * P,
                   jnp.where(lane < 12, P,
                             jnp.where(lane == 12, 1,
                                       jnp.where(lane < 15, 2, 0))))
    b3 = jnp.where(lane < 8, PA_OFF + 2 * L + lane,
                   jnp.where(lane < 12, 8 + lane,
                             jnp.where(lane == 12, CP_OFF,
                                       jnp.where(lane == 13, CA_OFF,
                                                 jnp.where(lane == 14, CA_OFF + 1,
                                                           K_OFF)))))
    o3 = jnp.where(lane < 8, alt,
                   jnp.where(lane < 12, zl,
                             jnp.where(lane == 12, NPRED + 2 * NARG,
                                       jnp.where(lane == 13, 2 * NPRED + 2 * NARG,
                                                 jnp.where(lane == 14, 2 * NPRED + 3 * NARG,
                                                           zl)))))
    mbo = (
        (jnp.full((L,), P, jnp.int32), PP_OFF + lane, zl),
        (jnp.full((L,), 2 * P, jnp.int32), PA_OFF + lane, alt),
        (jnp.full((L,), 2 * P, jnp.int32), PA_OFF + L + lane, alt),
        (m3, b3, o3),
    )
    for k in range(NIDX // L):
        for t in range(3):
            idx_v[pl.ds(MBO_OFF + (3 * k + t) * L, L)] = mbo[k][t]

    col = [lane + (L * c) for c in range(NCH)]
    w2u = [table_v[pl.ds(RW2 * DW + L * c, L)] for c in range(NCH)]
    b2lo, b2hi = plsc.unpack(
        plsc.bitcast(table_v[pl.ds(RB2 * DW, L)], jnp.bfloat16),
        format=plsc.PackFormat.INTERLEAVED)
    b2vec = jnp.full((L,), jnp.sum(b2lo + b2hi))
    zero32 = jnp.zeros((2 * L,), jnp.bfloat16)

    def body(b, carry):
        bvec = jnp.full((L,), b, jnp.int32)

        def chunk(k, accs):
            accs = list(accs)
            mk = idx_v[pl.ds(MBO_OFF + (3 * k + 0) * L, L)]
            bk = idx_v[pl.ds(MBO_OFF + (3 * k + 1) * L, L)]
            ok = idx_v[pl.ds(MBO_OFF + (3 * k + 2) * L, L)]
            raw = plsc.load_gather(idx_v, [bvec * mk + bk])
            addr = (raw + ok) * DW
            for j in range(L):
                r = jnp.full((L,), addr[j], jnp.int32)
                for c in range(NCH):
                    w = plsc.load_gather(table_v, [r + col[c]])
                    accs[c] = accs[c] + plsc.bitcast(w, jnp.bfloat16)
            return tuple(accs)

        accs = lax.fori_loop(0, NIDX // L, chunk, (zero32,) * NCH)
        for c in range(NCH):
            hacc_v[pl.ds(b * DW + L * c, L)] = plsc.bitcast(accs[c], jnp.int32)
        return carry

    lax.fori_loop(0, BPW, body, 0)

    rowoff = lane * DW

    def epi16(g, carry):
        gbase = g * (L * DW)

        def wstep(w):
            hw = plsc.load_gather(hacc_v, [rowoff + (gbase + w)])
            h32 = plsc.bitcast(hw, jnp.bfloat16)
            w2w = plsc.bitcast(jnp.full((L,), w2u[w // L][w % L], jnp.int32),
                               jnp.bfloat16)
            return jnp.maximum(h32, 0) * w2w

        acc32 = zero32
        for w in range(DW):
            acc32 = acc32 + wstep(w)
        lo, hi = plsc.unpack(acc32, format=plsc.PackFormat.INTERLEAVED)
        tot = lo + hi + b2vec
        out_v[pl.ds(g * L, L)] = 1.0 / (1.0 + jnp.exp(-tot))
        return carry

    lax.fori_loop(0, BPW // L, epi16, 0)
    pltpu.sync_copy(out_v, out_hbm.at[pl.ds(base, BPW)])


def kernel(prem_pred_idx, prem_arg_idx, concl_pred_idx, concl_arg_idx,
           pred_table, arg_table, W1, b1, W2, b2):
    pp = prem_pred_idx.astype(jnp.int32).reshape(-1)
    pa = prem_arg_idx.astype(jnp.int32).reshape(-1)
    cp = concl_pred_idx.astype(jnp.int32).reshape(-1)
    ca = concl_arg_idx.astype(jnp.int32).reshape(-1)

    folded = pl.pallas_call(
        _fold_body,
        out_shape=jax.ShapeDtypeStruct((NROWS, D), jnp.float32),
    )(pred_table, arg_table, W1, b1, W2, b2)

    out_flat = _sc_gather(_pack_pairs(folded).reshape(-1), pp, pa, cp, ca)
    return out_flat.reshape(B, 1)


# speedup vs baseline: 1.6551x; 1.6551x over previous
import functools

import jax
import jax.numpy as jnp
from jax import lax
from jax.experimental import pallas as pl
from jax.experimental.pallas import tpu as pltpu
from jax.experimental.pallas import tpu_sc as plsc

B = 16384
P = 20
D = 128
NPRED = 64
NARG = 128
NROWS = 648
RB1 = 640
RW2 = 641
RB2 = 642
NIDX = 64
NW = 32
BPW = B // NW
L = 16
DW = D // 2
NCH = D // (2 * L)


def _fold_body(pred_ref, arg_ref, w1_ref, b1_ref, w2_ref, b2_ref, out_ref):
    pred = pred_ref[...]
    arg = arg_ref[...]
    w1 = w1_ref[...]
    s = jnp.float32(1.0 / P)
    b2row = jnp.where(
        lax.broadcasted_iota(jnp.int32, (1, D), 1) == 0, b2_ref[...][0], 0.0)
    parts = [
        jnp.dot(pred, w1[0 * D:1 * D], preferred_element_type=jnp.float32) * s,
        jnp.dot(arg, w1[1 * D:2 * D], preferred_element_type=jnp.float32) * s,
        jnp.dot(arg, w1[2 * D:3 * D], preferred_element_type=jnp.float32) * s,
        jnp.dot(pred, w1[3 * D:4 * D], preferred_element_type=jnp.float32),
        jnp.dot(arg, w1[4 * D:5 * D], preferred_element_type=jnp.float32),
        jnp.dot(arg, w1[5 * D:6 * D], preferred_element_type=jnp.float32),
        b1_ref[...][None, :],
        w2_ref[...][:, 0][None, :],
        b2row,
        jnp.zeros((NROWS - RB2 - 1, D), jnp.float32),
    ]
    out_ref[...] = jnp.concatenate(parts, axis=0)


def _pack_pairs(x_f32):
    xb = x_f32.astype(jnp.bfloat16)
    n = xb.shape[-1] // 2
    lo = lax.bitcast_convert_type(xb[:, :n], jnp.uint16).astype(jnp.uint32)
    hi = lax.bitcast_convert_type(xb[:, n:], jnp.uint16).astype(jnp.uint32)
    return lax.bitcast_convert_type(lo | (hi << 16), jnp.int32)


@functools.partial(
    pl.kernel,
    mesh=plsc.VectorSubcoreMesh(core_axis_name="c", subcore_axis_name="s"),
    out_type=jax.ShapeDtypeStruct((B,), jnp.float32),
    compiler_params=pltpu.CompilerParams(needs_layout_passes=False),
    scratch_types=[
        pltpu.VMEM((NROWS * DW,), jnp.int32),
        pltpu.VMEM((NIDX, BPW), jnp.int32),
        pltpu.VMEM((BPW * DW,), jnp.int32),
        pltpu.VMEM((BPW,), jnp.float32),
    ],
)
def _sc_gather(table_hbm, cidx_hbm, out_hbm, table_v, cidx_v, hacc_v, out_v):
    wid = lax.axis_index("s") * 2 + lax.axis_index("c")
    base = wid * BPW
    pltpu.sync_copy(table_hbm, table_v)
    pltpu.sync_copy(cidx_hbm.at[:, pl.ds(base, BPW)], cidx_v)

    col = [lax.iota(jnp.int32, L) + (L * c) for c in range(NCH)]
    kvec = [lax.iota(jnp.int32, L) + (L * k) for k in range(NIDX // L)]
    w2u = [table_v[pl.ds(RW2 * DW + L * c, L)] for c in range(NCH)]
    b2lo, b2hi = plsc.unpack(
        plsc.bitcast(table_v[pl.ds(RB2 * DW, L)], jnp.bfloat16),
        format=plsc.PackFormat.INTERLEAVED)
    b2vec = jnp.full((L,), jnp.sum(b2lo + b2hi))
    zero32 = jnp.zeros((2 * L,), jnp.bfloat16)

    def body(b, carry):
        bvec = jnp.full((L,), b, jnp.int32)

        def chunk(k, accs):
            accs = list(accs)
            kv = jnp.where(k == 0, kvec[0],
                           jnp.where(k == 1, kvec[1],
                                     jnp.where(k == 2, kvec[2], kvec[3])))
            iv = plsc.load_gather(cidx_v, [kv, bvec])
            for j in range(L):
                r = jnp.full((L,), iv[j], jnp.int32)
                for c in range(NCH):
                    w = plsc.load_gather(table_v, [r + col[c]])
                    accs[c] = accs[c] + plsc.bitcast(w, jnp.bfloat16)
            return tuple(accs)

        accs = lax.fori_loop(0, NIDX // L, chunk, (zero32,) * NCH)
        for c in range(NCH):
            hacc_v[pl.ds(b * DW + L * c, L)] = plsc.bitcast(accs[c], jnp.int32)
        return carry

    lax.fori_loop(0, BPW, body, 0)

    rowoff = lax.iota(jnp.int32, L) * DW

    def epi16(g, carry):
        gbase = g * (L * DW)

        def wstep(w):
            hw = plsc.load_gather(hacc_v, [rowoff + (gbase + w)])
            h32 = plsc.bitcast(hw, jnp.bfloat16)
            w2w = plsc.bitcast(jnp.full((L,), w2u[w // L][w % L], jnp.int32),
                               jnp.bfloat16)
            return jnp.maximum(h32, 0) * w2w

        acc32 = zero32
        for w in range(DW):
            acc32 = acc32 + wstep(w)
        lo, hi = plsc.unpack(acc32, format=plsc.PackFormat.INTERLEAVED)
        tot = lo + hi + b2vec
        out_v[pl.ds(g * L, L)] = 1.0 / (1.0 + jnp.exp(-tot))
        return carry

    lax.fori_loop(0, BPW // L, epi16, 0)
    pltpu.sync_copy(out_v, out_hbm.at[pl.ds(base, BPW)])


def kernel(prem_pred_idx, prem_arg_idx, concl_pred_idx, concl_arg_idx,
           pred_table, arg_table, W1, b1, W2, b2):
    pp = prem_pred_idx.astype(jnp.int32)
    pa = prem_arg_idx.astype(jnp.int32)
    cp = concl_pred_idx.astype(jnp.int32)
    ca = concl_arg_idx.astype(jnp.int32)

    folded = pl.pallas_call(
        _fold_body,
        out_shape=jax.ShapeDtypeStruct((NROWS, D), jnp.float32),
    )(pred_table, arg_table, W1, b1, W2, b2)

    cidx_t = jnp.concatenate([
        pp.T,
        pa[:, :, 0].T + NPRED,
        pa[:, :, 1].T + (NPRED + NARG),
        cp[None, :] + (NPRED + 2 * NARG),
        ca[:, 0][None, :] + (2 * NPRED + 2 * NARG),
        ca[:, 1][None, :] + (2 * NPRED + 3 * NARG),
        jnp.full((1, B), RB1, jnp.int32),
    ], axis=0) * DW

    out_flat = _sc_gather(_pack_pairs(folded).reshape(-1), cidx_t)
    return out_flat.reshape(B, 1)
